# Initial kernel scaffold; baseline (speedup 1.0000x reference)
import jax
import jax.numpy as jnp
from jax.experimental import pallas as pl


def kernel(merchant_ids, embedding_table):
    raise NotImplementedError("write your pallas kernel here")


# speedup vs baseline: 7.5027x; 7.5027x over previous
import functools

import jax
import jax.numpy as jnp
from jax import lax
from jax.experimental import pallas as pl
from jax.experimental.pallas import tpu as pltpu
from jax.experimental.pallas import tpu_sc as plsc

BATCH = 16384
HIST = 50
D = 32
VOCAB1 = 100001
NC = 2
NS = 16
NW = NC * NS
BCHUNK = 8192
NBCH = BATCH // BCHUNK
L = 16

_MESH = plsc.VectorSubcoreMesh(core_axis_name="c", subcore_axis_name="s")


@functools.partial(
    pl.kernel,
    mesh=_MESH,
    out_type=jax.ShapeDtypeStruct((HIST, D, BATCH), jnp.float32),
    scratch_types=[
        pltpu.VMEM((VOCAB1,), jnp.float32),
        pltpu.VMEM((BCHUNK,), jnp.int32),
        pltpu.VMEM((BCHUNK,), jnp.float32),
    ],
    compiler_params=pltpu.CompilerParams(needs_layout_passes=False),
)
def _gather_kernel(ids_hbm, table_hbm, out_hbm, plane_v, idx_v, row_v):
    wid = lax.axis_index("s") * NC + lax.axis_index("c")

    pltpu.sync_copy(table_hbm.at[wid], plane_v)

    def chunk_body(hb, carry):
        h = hb // NBCH
        b0 = (hb % NBCH) * BCHUNK
        pltpu.sync_copy(ids_hbm.at[h, pl.ds(b0, BCHUNK)], idx_v)

        def gather16(j, c2):
            base = j * L
            idx16 = idx_v[pl.ds(base, L)]
            row_v[pl.ds(base, L)] = plsc.load_gather(plane_v, [idx16])
            return c2

        lax.fori_loop(0, BCHUNK // L, gather16, 0)
        pltpu.sync_copy(row_v, out_hbm.at[h, wid, pl.ds(b0, BCHUNK)])
        return carry

    lax.fori_loop(0, HIST * NBCH, chunk_body, 0)


def kernel(merchant_ids, embedding_table):
    ids_t = merchant_ids.T.astype(jnp.int32)
    table_t = embedding_table.T
    out_t = _gather_kernel(ids_t, table_t)
    return out_t.transpose(2, 0, 1)


# speedup vs baseline: 13.9513x; 1.8595x over previous
import functools

import jax
import jax.numpy as jnp
from jax import lax
from jax.experimental import pallas as pl
from jax.experimental.pallas import tpu as pltpu
from jax.experimental.pallas import tpu_sc as plsc

BATCH = 16384
HIST = 50
D = 32
VOCAB1 = 100001
NC = 2
NS = 16
NW = NC * NS
BCHUNK = 4096
NBCH = BATCH // BCHUNK
NCHT = HIST * NBCH
L = 16
UNROLL = 8

_MESH = plsc.VectorSubcoreMesh(core_axis_name="c", subcore_axis_name="s")


@functools.partial(
    pl.kernel,
    mesh=_MESH,
    out_type=jax.ShapeDtypeStruct((HIST, D, BATCH), jnp.float32),
    scratch_types=[
        pltpu.VMEM((VOCAB1,), jnp.float32),
        pltpu.VMEM((BCHUNK,), jnp.int32),
        pltpu.VMEM((BCHUNK,), jnp.int32),
        pltpu.VMEM((BCHUNK,), jnp.float32),
        pltpu.VMEM((BCHUNK,), jnp.float32),
        pltpu.SemaphoreType.DMA,
        pltpu.SemaphoreType.DMA,
        pltpu.SemaphoreType.DMA,
        pltpu.SemaphoreType.DMA,
    ],
    compiler_params=pltpu.CompilerParams(needs_layout_passes=False),
)
def _gather_kernel(ids_hbm, table_hbm, out_hbm, plane_v,
                   idx_a, idx_b, row_a, row_b, isem_a, isem_b, osem_a, osem_b):
    wid = lax.axis_index("s") * NC + lax.axis_index("c")

    pltpu.sync_copy(table_hbm.at[wid], plane_v)

    def start_idx(t, idx_v, isem):
        h = t // NBCH
        b0 = (t % NBCH) * BCHUNK
        pltpu.async_copy(ids_hbm.at[h, pl.ds(b0, BCHUNK)], idx_v, isem)

    def wait_idx(idx_v, isem):
        pltpu.make_async_copy(ids_hbm.at[0, pl.ds(0, BCHUNK)], idx_v, isem).wait()

    def start_out(t, row_v, osem):
        h = t // NBCH
        b0 = (t % NBCH) * BCHUNK
        pltpu.async_copy(row_v, out_hbm.at[h, wid, pl.ds(b0, BCHUNK)], osem)

    def wait_out(row_v, osem):
        pltpu.make_async_copy(row_v, out_hbm.at[0, wid, pl.ds(0, BCHUNK)], osem).wait()

    def gather(idx_v, row_v):
        def body(j, c2):
            base0 = j * (L * UNROLL)
            for k in range(UNROLL):
                base = base0 + k * L
                idx16 = idx_v[pl.ds(base, L)]
                row_v[pl.ds(base, L)] = plsc.load_gather(plane_v, [idx16])
            return c2

        lax.fori_loop(0, BCHUNK // (L * UNROLL), body, 0)

    start_idx(0, idx_a, isem_a)
    start_idx(1, idx_b, isem_b)

    def pair_body(i2, carry):
        t = i2 * 2
        wait_idx(idx_a, isem_a)

        @pl.when(t >= 2)
        def _():
            wait_out(row_a, osem_a)

        gather(idx_a, row_a)
        start_out(t, row_a, osem_a)

        @pl.when(t + 2 < NCHT)
        def _():
            start_idx(t + 2, idx_a, isem_a)

        wait_idx(idx_b, isem_b)

        @pl.when(t >= 2)
        def _():
            wait_out(row_b, osem_b)

        gather(idx_b, row_b)
        start_out(t + 1, row_b, osem_b)

        @pl.when(t + 3 < NCHT)
        def _():
            start_idx(t + 3, idx_b, isem_b)

        return carry

    lax.fori_loop(0, NCHT // 2, pair_body, 0)
    wait_out(row_a, osem_a)
    wait_out(row_b, osem_b)


def kernel(merchant_ids, embedding_table):
    ids_t = merchant_ids.T.astype(jnp.int32)
    table_t = embedding_table.T
    out_t = _gather_kernel(ids_t, table_t)
    return out_t.transpose(2, 0, 1)


# speedup vs baseline: 19.4065x; 1.3910x over previous
import functools

import jax
import jax.numpy as jnp
from jax import lax
from jax.experimental import pallas as pl
from jax.experimental.pallas import tpu as pltpu
from jax.experimental.pallas import tpu_sc as plsc

BATCH = 16384
HIST = 50
D = 32
VOCAB1 = 100001
NC = 2
NS = 16
NW = NC * NS
BCHUNK = 4096
NBCH = BATCH // BCHUNK
NCHT = HIST * NBCH
L = 16
UNROLL = 8

_MESH = plsc.VectorSubcoreMesh(core_axis_name="c", subcore_axis_name="s")


@functools.partial(
    pl.kernel,
    mesh=_MESH,
    out_type=jax.ShapeDtypeStruct((HIST, D, BATCH), jnp.float32),
    scratch_types=[
        pltpu.VMEM((VOCAB1,), jnp.float32),
        pltpu.VMEM((BCHUNK,), jnp.int32),
        pltpu.VMEM((BCHUNK,), jnp.int32),
        pltpu.VMEM((BCHUNK,), jnp.float32),
        pltpu.VMEM((BCHUNK,), jnp.float32),
        pltpu.SemaphoreType.DMA,
        pltpu.SemaphoreType.DMA,
        pltpu.SemaphoreType.DMA,
        pltpu.SemaphoreType.DMA,
    ],
    compiler_params=pltpu.CompilerParams(needs_layout_passes=False),
)
def _gather_kernel(ids_hbm, table_hbm, out_hbm, plane_v,
                   idx_a, idx_b, row_a, row_b, isem_a, isem_b, osem_a, osem_b):
    wid = lax.axis_index("s") * NC + lax.axis_index("c")

    pltpu.sync_copy(table_hbm.at[wid], plane_v)

    def start_idx(t, idx_v, isem):
        h = t // NBCH
        b0 = (t % NBCH) * BCHUNK
        pltpu.async_copy(ids_hbm.at[h, pl.ds(b0, BCHUNK)], idx_v, isem)

    def wait_idx(idx_v, isem):
        pltpu.make_async_copy(ids_hbm.at[0, pl.ds(0, BCHUNK)], idx_v, isem).wait()

    def start_out(t, row_v, osem):
        h = t // NBCH
        b0 = (t % NBCH) * BCHUNK
        pltpu.async_copy(row_v, out_hbm.at[h, wid, pl.ds(b0, BCHUNK)], osem)

    def wait_out(row_v, osem):
        pltpu.make_async_copy(row_v, out_hbm.at[0, wid, pl.ds(0, BCHUNK)], osem).wait()

    def gather(idx_v, row_v):
        @plsc.parallel_loop(0, BCHUNK // L, unroll=UNROLL)
        def body(j):
            base = j * L
            idx16 = idx_v[pl.ds(base, L)]
            row_v[pl.ds(base, L)] = plsc.load_gather(plane_v, [idx16])

    start_idx(0, idx_a, isem_a)
    start_idx(1, idx_b, isem_b)

    def pair_body(i2, carry):
        t = i2 * 2
        wait_idx(idx_a, isem_a)

        @pl.when(t >= 2)
        def _():
            wait_out(row_a, osem_a)

        gather(idx_a, row_a)
        start_out(t, row_a, osem_a)

        @pl.when(t + 2 < NCHT)
        def _():
            start_idx(t + 2, idx_a, isem_a)

        wait_idx(idx_b, isem_b)

        @pl.when(t >= 2)
        def _():
            wait_out(row_b, osem_b)

        gather(idx_b, row_b)
        start_out(t + 1, row_b, osem_b)

        @pl.when(t + 3 < NCHT)
        def _():
            start_idx(t + 3, idx_b, isem_b)

        return carry

    lax.fori_loop(0, NCHT // 2, pair_body, 0)
    wait_out(row_a, osem_a)
    wait_out(row_b, osem_b)


def kernel(merchant_ids, embedding_table):
    ids_t = merchant_ids.T.astype(jnp.int32)
    table_t = embedding_table.T
    out_t = _gather_kernel(ids_t, table_t)
    return out_t.transpose(2, 0, 1)
